# Initial kernel scaffold; baseline (speedup 1.0000x reference)
import jax
import jax.numpy as jnp
from jax.experimental import pallas as pl


def kernel(x, edge_index0, edge_type0, edge_index1, edge_type1, W1, b1, W2, b2):
    raise NotImplementedError("write your pallas kernel here")


# speedup vs baseline: 6.9624x; 6.9624x over previous
import functools

import jax
import jax.numpy as jnp
from jax import lax
from jax.experimental import pallas as pl
from jax.experimental.pallas import tpu as pltpu
from jax.experimental.pallas import tpu_sc as plsc

N = 10000
F = 128
NREL = 3
RN = NREL * N
RNP = 30720
NA = 10240
NC = 2
NS = 16
NW = NC * NS
C = 128
BN = 1000
EP0 = 323584
EP1 = 163840


def _sc_aggregate(ep):
  n_deg = ep // NS // C
  n_main = ep // NW // C
  dslice = RNP // NS
  aslice = NA // NS
  mesh = plsc.VectorSubcoreMesh(
      core_axis_name="c", subcore_axis_name="s", num_cores=NC,
      num_subcores=NS)

  @functools.partial(
      pl.kernel,
      out_type=jax.ShapeDtypeStruct((NC, NA, F), jnp.float32),
      mesh=mesh,
      compiler_params=pltpu.CompilerParams(needs_layout_passes=False),
      scratch_types=[
          pltpu.VMEM_SHARED((RNP,), jnp.float32),
          pltpu.VMEM_SHARED((NA, F), jnp.float32),
          pltpu.VMEM((dslice,), jnp.float32),
          pltpu.VMEM((C, F), jnp.float32),
          pltpu.VMEM((C,), jnp.int32),
          pltpu.VMEM((C,), jnp.int32),
          pltpu.VMEM((C,), jnp.int32),
          pltpu.VMEM((C,), jnp.int32),
          pltpu.VMEM((C,), jnp.int32),
          pltpu.VMEM((C,), jnp.float32),
          pltpu.VMEM((C,), jnp.float32),
          pltpu.SemaphoreType.DMA,
      ],
  )
  def agg(y_hbm, src_hbm, dst_hbm, et_hbm, zacc_hbm, zdeg_hbm, out_hbm,
          deg_sh, acc_sh, wv_v, rows_v, src_v, dst_v, et_v, gk_v,
          k_v, w_v, ones_v, sem):
    cid = lax.axis_index("c")
    sid = lax.axis_index("s")
    wid = sid * NC + cid

    for j in range(C // 16):
      ones_v[pl.ds(j * 16, 16)] = jnp.ones((16,), jnp.float32)

    pltpu.sync_copy(zdeg_hbm.at[pl.ds(sid * dslice, dslice)],
                    deg_sh.at[pl.ds(sid * dslice, dslice)])
    pltpu.sync_copy(zacc_hbm.at[pl.ds(sid * aslice, aslice)],
                    acc_sh.at[pl.ds(sid * aslice, aslice)])
    plsc.subcore_barrier()

    dbase = sid * (ep // NS)
    def deg_chunk(g, carry):
      eb = dbase + g * C
      pltpu.sync_copy(dst_hbm.at[pl.ds(eb, C)], dst_v)
      pltpu.sync_copy(et_hbm.at[pl.ds(eb, C)], et_v)
      for j in range(C // 16):
        sl = pl.ds(j * 16, 16)
        k_v[sl] = et_v[sl] * N + dst_v[sl]
      pltpu.sync_copy(ones_v, deg_sh.at[k_v], add=True)
      return carry
    lax.fori_loop(0, n_deg, deg_chunk, 0)
    plsc.subcore_barrier()

    wsl = pl.ds(sid * dslice, dslice)
    pltpu.sync_copy(deg_sh.at[wsl], wv_v)
    for j in range(dslice // 16):
      sl = pl.ds(j * 16, 16)
      wv_v[sl] = 1.0 / jnp.maximum(wv_v[sl], 1.0)
    pltpu.sync_copy(wv_v, deg_sh.at[wsl])
    plsc.subcore_barrier()

    mbase = wid * (ep // NW)
    def main_chunk(g, carry):
      eb = mbase + g * C
      pltpu.sync_copy(src_hbm.at[pl.ds(eb, C)], src_v)
      pltpu.sync_copy(dst_hbm.at[pl.ds(eb, C)], dst_v)
      pltpu.sync_copy(et_hbm.at[pl.ds(eb, C)], et_v)
      for j in range(C // 16):
        sl = pl.ds(j * 16, 16)
        rel = et_v[sl] * N
        gk_v[sl] = rel + src_v[sl]
        k_v[sl] = rel + dst_v[sl]
      pltpu.async_copy(y_hbm.at[gk_v], rows_v, sem).wait()
      pltpu.sync_copy(deg_sh.at[k_v], w_v)
      def scale(g, c2):
        w16 = w_v[pl.ds(g * 16, 16)]
        for i in range(16):
          w = w16[i]
          for j in range(F // 16):
            sl = pl.ds(j * 16, 16)
            rows_v[g * 16 + i, sl] = rows_v[g * 16 + i, sl] * w
        return c2
      lax.fori_loop(0, C // 16, scale, 0)
      pltpu.sync_copy(rows_v, acc_sh.at[dst_v], add=True)
      return carry
    lax.fori_loop(0, n_main, main_chunk, 0)
    plsc.subcore_barrier()

    osl = pl.ds(sid * aslice, aslice)
    pltpu.sync_copy(acc_sh.at[osl], out_hbm.at[cid, osl])

  return agg


def _mm1(x, w):
  def body(x_ref, w_ref, o_ref):
    o_ref[0] = jnp.dot(x_ref[...], w_ref[0], preferred_element_type=jnp.float32)
  return pl.pallas_call(
      body,
      grid=(NREL, N // BN),
      in_specs=[pl.BlockSpec((BN, F), lambda r, n: (n, 0)),
                pl.BlockSpec((1, F, F), lambda r, n: (r, 0, 0))],
      out_specs=pl.BlockSpec((1, BN, F), lambda r, n: (r, n, 0)),
      out_shape=jax.ShapeDtypeStruct((NREL, N, F), jnp.float32),
  )(x, w)


def _mm2(hp, b1, w2):
  def body(hp_ref, b_ref, w_ref, o_ref):
    h = hp_ref[0] + hp_ref[1] + jnp.sum(b_ref[...], axis=0, keepdims=True)
    o_ref[0] = jnp.dot(h, w_ref[0], preferred_element_type=jnp.float32)
  return pl.pallas_call(
      body,
      grid=(NREL, N // BN),
      in_specs=[pl.BlockSpec((2, BN, F), lambda r, n: (0, n, 0)),
                pl.BlockSpec((NREL, F), lambda r, n: (0, 0)),
                pl.BlockSpec((1, F, F), lambda r, n: (r, 0, 0))],
      out_specs=pl.BlockSpec((1, BN, F), lambda r, n: (r, n, 0)),
      out_shape=jax.ShapeDtypeStruct((NREL, N, F), jnp.float32),
  )(hp, b1, w2)


def _final(op, b2):
  def body(op_ref, b_ref, o_ref):
    o_ref[...] = op_ref[0] + op_ref[1] + jnp.sum(b_ref[...], axis=0,
                                                 keepdims=True)
  return pl.pallas_call(
      body,
      grid=(N // BN,),
      in_specs=[pl.BlockSpec((2, BN, F), lambda n: (0, n, 0)),
                pl.BlockSpec((NREL, F), lambda n: (0, 0))],
      out_specs=pl.BlockSpec((BN, F), lambda n: (n, 0)),
      out_shape=jax.ShapeDtypeStruct((N, F), jnp.float32),
  )(op, b2)


def _pad_edges(src, dst, et, ep):
  p = ep - src.shape[0]
  i = jnp.arange(p, dtype=jnp.int32)
  src = jnp.concatenate([src, i % 4096])
  dst = jnp.concatenate([dst, N + (i % 16)])
  et = jnp.concatenate([et, jnp.full((p,), NREL - 1, jnp.int32)])
  return src, dst, et


def kernel(x, edge_index0, edge_type0, edge_index1, edge_type1, W1, b1,
           W2, b2):
  z_acc = jnp.zeros((NA, F), jnp.float32)
  z_deg = jnp.zeros((RNP,), jnp.float32)

  y1 = _mm1(x, W1).reshape(RN, F)
  s0, d0, t0 = _pad_edges(edge_index0[0], edge_index0[1], edge_type0, EP0)
  hp = _sc_aggregate(EP0)(y1, s0, d0, t0, z_acc, z_deg)

  y2 = _mm2(hp, b1, W2).reshape(RN, F)
  s1, d1, t1 = _pad_edges(edge_index1[0], edge_index1[1], edge_type1, EP1)
  op = _sc_aggregate(EP1)(y2, s1, d1, t1, z_acc, z_deg)

  return _final(op, b2)


# speedup vs baseline: 13.7246x; 1.9712x over previous
import functools

import jax
import jax.numpy as jnp
from jax import lax
from jax.experimental import pallas as pl
from jax.experimental.pallas import tpu as pltpu
from jax.experimental.pallas import tpu_sc as plsc

N = 10000
F = 128
NREL = 3
RN = NREL * N
RNP = 30720
NA = 10112
CD = 256
NC = 2
NS = 16
NW = NC * NS
C = 128
BN = 1000
EP0 = 323584
EP1 = 163840


def _sc_aggregate(ep):
  n_deg = ep // NS // CD
  n_main = ep // NW // C
  dslice = RNP // NS
  aslice = NA // NS
  mesh = plsc.VectorSubcoreMesh(
      core_axis_name="c", subcore_axis_name="s", num_cores=NC,
      num_subcores=NS)

  @functools.partial(
      pl.kernel,
      out_type=jax.ShapeDtypeStruct((NC, NA, F), jnp.float32),
      mesh=mesh,
      compiler_params=pltpu.CompilerParams(needs_layout_passes=False),
      scratch_types=[
          pltpu.VMEM_SHARED((RNP,), jnp.float32),
          pltpu.VMEM_SHARED((NA, F), jnp.float32),
          pltpu.VMEM((dslice,), jnp.float32),
          pltpu.VMEM((2, C, F), jnp.float32),
          pltpu.VMEM((C,), jnp.int32),
          pltpu.VMEM((C,), jnp.int32),
          pltpu.VMEM((2, C), jnp.int32),
          pltpu.VMEM((2, C), jnp.int32),
          pltpu.VMEM((2, C), jnp.int32),
          pltpu.VMEM((2, C), jnp.float32),
          pltpu.VMEM((CD,), jnp.int32),
          pltpu.VMEM((CD,), jnp.int32),
          pltpu.VMEM((CD,), jnp.int32),
          pltpu.VMEM((CD,), jnp.float32),
          pltpu.SemaphoreType.DMA,
          pltpu.SemaphoreType.DMA,
          pltpu.SemaphoreType.DMA,
          pltpu.SemaphoreType.DMA,
          pltpu.SemaphoreType.DMA,
      ],
  )
  def agg(y_hbm, src_hbm, dst_hbm, et_hbm, zacc_hbm, zdeg_hbm, out_hbm,
          deg_sh, acc_sh, wv_v, rows_v, src_v, et_v, dst_v, gk_v, k_v,
          w_v, dd_v, de_v, dk_v, ones_v, sem_i, sem_r0, sem_r1, sem_w0,
          sem_w1):
    cid = lax.axis_index("c")
    sid = lax.axis_index("s")
    wid = sid * NC + cid
    sem_r = (sem_r0, sem_r1)
    sem_w = (sem_w0, sem_w1)

    for j in range(CD // 16):
      ones_v[pl.ds(j * 16, 16)] = jnp.ones((16,), jnp.float32)

    pltpu.sync_copy(zdeg_hbm.at[pl.ds(sid * dslice, dslice)],
                    deg_sh.at[pl.ds(sid * dslice, dslice)])
    pltpu.sync_copy(zacc_hbm.at[pl.ds(sid * aslice, aslice)],
                    acc_sh.at[pl.ds(sid * aslice, aslice)])
    plsc.subcore_barrier()

    dbase = sid * (ep // NS)
    def deg_chunk(g, carry):
      eb = dbase + g * CD
      c1 = pltpu.async_copy(dst_hbm.at[pl.ds(eb, CD)], dd_v, sem_i)
      c2 = pltpu.async_copy(et_hbm.at[pl.ds(eb, CD)], de_v, sem_i)
      c1.wait()
      c2.wait()
      for j in range(CD // 16):
        sl = pl.ds(j * 16, 16)
        dk_v[sl] = de_v[sl] * N + dd_v[sl]
      pltpu.sync_copy(ones_v, deg_sh.at[dk_v], add=True)
      return carry
    lax.fori_loop(0, n_deg, deg_chunk, 0)
    plsc.subcore_barrier()

    wsl = pl.ds(sid * dslice, dslice)
    pltpu.sync_copy(deg_sh.at[wsl], wv_v)
    for j in range(dslice // 16):
      sl = pl.ds(j * 16, 16)
      wv_v[sl] = 1.0 / jnp.maximum(wv_v[sl], 1.0)
    pltpu.sync_copy(wv_v, deg_sh.at[wsl])
    plsc.subcore_barrier()

    mbase = wid * (ep // NW)

    def fetch_issue(g, s):
      eb = mbase + g * C
      c1 = pltpu.async_copy(src_hbm.at[pl.ds(eb, C)], src_v, sem_i)
      c2 = pltpu.async_copy(dst_hbm.at[pl.ds(eb, C)], dst_v.at[s], sem_i)
      c3 = pltpu.async_copy(et_hbm.at[pl.ds(eb, C)], et_v, sem_i)
      c1.wait()
      c2.wait()
      c3.wait()
      for j in range(C // 16):
        sl = pl.ds(j * 16, 16)
        rel = et_v[sl] * N
        gk_v[s, sl] = rel + src_v[sl]
        k_v[s, sl] = rel + dst_v[s, sl]
      pltpu.async_copy(y_hbm.at[gk_v.at[s]], rows_v.at[s], sem_r[s])
      pltpu.async_copy(deg_sh.at[k_v.at[s]], w_v.at[s], sem_w[s])

    def finish(s):
      pltpu.make_async_copy(y_hbm.at[gk_v.at[s]], rows_v.at[s],
                            sem_r[s]).wait()
      pltpu.make_async_copy(deg_sh.at[k_v.at[s]], w_v.at[s],
                            sem_w[s]).wait()
      def scale(gr, c2):
        w16 = w_v[s, pl.ds(gr * 16, 16)]
        for i in range(16):
          w = w16[i]
          for j in range(F // 16):
            sl = pl.ds(j * 16, 16)
            rows_v[s, gr * 16 + i, sl] = rows_v[s, gr * 16 + i, sl] * w
        return c2
      lax.fori_loop(0, C // 16, scale, 0)
      pltpu.sync_copy(rows_v.at[s], acc_sh.at[dst_v.at[s]], add=True)

    fetch_issue(0, 0)
    def pair(h, carry):
      g = 2 * h
      fetch_issue(g + 1, 1)
      finish(0)
      @pl.when(g + 2 < n_main)
      def _():
        fetch_issue(g + 2, 0)
      finish(1)
      return carry
    lax.fori_loop(0, n_main // 2, pair, 0)
    if n_main % 2:
      finish(0)
    plsc.subcore_barrier()

    osl = pl.ds(sid * aslice, aslice)
    pltpu.sync_copy(acc_sh.at[osl], out_hbm.at[cid, osl])

  return agg


def _mm1(x, w):
  def body(x_ref, w_ref, o_ref):
    o_ref[0] = jnp.dot(x_ref[...], w_ref[0], preferred_element_type=jnp.float32)
  return pl.pallas_call(
      body,
      grid=(NREL, N // BN),
      in_specs=[pl.BlockSpec((BN, F), lambda r, n: (n, 0)),
                pl.BlockSpec((1, F, F), lambda r, n: (r, 0, 0))],
      out_specs=pl.BlockSpec((1, BN, F), lambda r, n: (r, n, 0)),
      out_shape=jax.ShapeDtypeStruct((NREL, N, F), jnp.float32),
  )(x, w)


def _mm2(hp, b1, w2):
  def body(hp_ref, b_ref, w_ref, o_ref):
    h = hp_ref[0] + hp_ref[1] + jnp.sum(b_ref[...], axis=0, keepdims=True)
    o_ref[0] = jnp.dot(h, w_ref[0], preferred_element_type=jnp.float32)
  return pl.pallas_call(
      body,
      grid=(NREL, N // BN),
      in_specs=[pl.BlockSpec((2, BN, F), lambda r, n: (0, n, 0)),
                pl.BlockSpec((NREL, F), lambda r, n: (0, 0)),
                pl.BlockSpec((1, F, F), lambda r, n: (r, 0, 0))],
      out_specs=pl.BlockSpec((1, BN, F), lambda r, n: (r, n, 0)),
      out_shape=jax.ShapeDtypeStruct((NREL, N, F), jnp.float32),
  )(hp, b1, w2)


def _final(op, b2):
  def body(op_ref, b_ref, o_ref):
    o_ref[...] = op_ref[0] + op_ref[1] + jnp.sum(b_ref[...], axis=0,
                                                 keepdims=True)
  return pl.pallas_call(
      body,
      grid=(N // BN,),
      in_specs=[pl.BlockSpec((2, BN, F), lambda n: (0, n, 0)),
                pl.BlockSpec((NREL, F), lambda n: (0, 0))],
      out_specs=pl.BlockSpec((BN, F), lambda n: (n, 0)),
      out_shape=jax.ShapeDtypeStruct((N, F), jnp.float32),
  )(op, b2)


def _pad_edges(src, dst, et, ep):
  p = ep - src.shape[0]
  i = jnp.arange(p, dtype=jnp.int32)
  src = jnp.concatenate([src, i % 4096])
  dst = jnp.concatenate([dst, N + (i % 16)])
  et = jnp.concatenate([et, jnp.full((p,), NREL - 1, jnp.int32)])
  return src, dst, et


def kernel(x, edge_index0, edge_type0, edge_index1, edge_type1, W1, b1,
           W2, b2):
  z_acc = jnp.zeros((NA, F), jnp.float32)
  z_deg = jnp.zeros((RNP,), jnp.float32)

  y1 = _mm1(x, W1).reshape(RN, F)
  s0, d0, t0 = _pad_edges(edge_index0[0], edge_index0[1], edge_type0, EP0)
  hp = _sc_aggregate(EP0)(y1, s0, d0, t0, z_acc, z_deg)

  y2 = _mm2(hp, b1, W2).reshape(RN, F)
  s1, d1, t1 = _pad_edges(edge_index1[0], edge_index1[1], edge_type1, EP1)
  op = _sc_aggregate(EP1)(y2, s1, d1, t1, z_acc, z_deg)

  return _final(op, b2)


# speedup vs baseline: 15.4882x; 1.1285x over previous
import functools

import jax
import jax.numpy as jnp
from jax import lax
from jax.experimental import pallas as pl
from jax.experimental.pallas import tpu as pltpu
from jax.experimental.pallas import tpu_sc as plsc

N = 10000
F = 128
NREL = 3
RN = NREL * N
RNP = 30720
NA = 10112
NC = 2
NS = 16
NW = NC * NS
C = 128
CD = 512
BN = 1000
EP0 = 327680
EP1 = 163840
DSL = RNP // NS
ASL = NA // NS

_MESH = dict(core_axis_name="c", subcore_axis_name="s", num_cores=NC,
             num_subcores=NS)
_PARAMS = pltpu.CompilerParams(needs_layout_passes=False)


def _sc_degrees():
  zsl = 2 * RNP // NS

  @functools.partial(
      pl.kernel,
      out_type=jax.ShapeDtypeStruct((2, NC, 1, RNP), jnp.float32),
      mesh=plsc.VectorSubcoreMesh(**_MESH),
      compiler_params=_PARAMS,
      scratch_types=[
          pltpu.VMEM_SHARED((2 * RNP,), jnp.float32),
          pltpu.VMEM((CD,), jnp.int32),
          pltpu.VMEM((CD,), jnp.int32),
          pltpu.VMEM((CD,), jnp.int32),
          pltpu.VMEM((CD,), jnp.float32),
          pltpu.SemaphoreType.DMA,
      ],
  )
  def deg(d0_hbm, t0_hbm, d1_hbm, t1_hbm, zdeg_hbm, out_hbm,
          deg_sh, dd_v, de_v, dk_v, ones_v, sem):
    cid = lax.axis_index("c")
    sid = lax.axis_index("s")
    wid = sid * NC + cid

    for j in range(CD // 16):
      ones_v[pl.ds(j * 16, 16)] = jnp.ones((16,), jnp.float32)
    pltpu.sync_copy(zdeg_hbm, deg_sh.at[pl.ds(sid * zsl, zsl)])
    plsc.subcore_barrier()

    for l, (dh, th, epl) in enumerate(((d0_hbm, t0_hbm, EP0),
                                       (d1_hbm, t1_hbm, EP1))):
      base = wid * (epl // NW)
      off = l * RNP
      def chunk(g, carry):
        eb = base + g * CD
        c1 = pltpu.async_copy(dh.at[pl.ds(eb, CD)], dd_v, sem)
        c2 = pltpu.async_copy(th.at[pl.ds(eb, CD)], de_v, sem)
        c1.wait()
        c2.wait()
        for j in range(CD // 16):
          sl = pl.ds(j * 16, 16)
          dk_v[sl] = de_v[sl] * N + dd_v[sl] + off
        pltpu.sync_copy(ones_v, deg_sh.at[dk_v], add=True)
        return carry
      lax.fori_loop(0, epl // NW // CD, chunk, 0)
    plsc.subcore_barrier()

    for l in range(2):
      pltpu.sync_copy(deg_sh.at[pl.ds(l * RNP + sid * DSL, DSL)],
                      out_hbm.at[l, cid, 0, pl.ds(sid * DSL, DSL)])

  return deg


def _winv(degs):
  def body(d_ref, o_ref):
    o_ref[0, 0] = 1.0 / jnp.maximum(d_ref[0, 0, 0] + d_ref[0, 1, 0], 1.0)
  return pl.pallas_call(
      body,
      grid=(2,),
      in_specs=[pl.BlockSpec((1, NC, 1, RNP), lambda l: (l, 0, 0, 0))],
      out_specs=pl.BlockSpec((1, 1, RNP), lambda l: (l, 0, 0)),
      out_shape=jax.ShapeDtypeStruct((2, 1, RNP), jnp.float32),
  )(degs)


def _sc_edges(ep, layer):
  n_main = ep // NW // C

  @functools.partial(
      pl.kernel,
      out_type=jax.ShapeDtypeStruct((NC, NA, F), jnp.float32),
      mesh=plsc.VectorSubcoreMesh(**_MESH),
      compiler_params=_PARAMS,
      scratch_types=[
          pltpu.VMEM_SHARED((RNP,), jnp.float32),
          pltpu.VMEM_SHARED((NA, F), jnp.float32),
          pltpu.VMEM((2, C, F), jnp.float32),
          pltpu.VMEM((C,), jnp.int32),
          pltpu.VMEM((C,), jnp.int32),
          pltpu.VMEM((2, C), jnp.int32),
          pltpu.VMEM((2, C), jnp.int32),
          pltpu.VMEM((2, C), jnp.int32),
          pltpu.VMEM((2, C), jnp.float32),
          pltpu.SemaphoreType.DMA,
          pltpu.SemaphoreType.DMA,
          pltpu.SemaphoreType.DMA,
          pltpu.SemaphoreType.DMA,
          pltpu.SemaphoreType.DMA,
      ],
  )
  def agg(y_hbm, src_hbm, dst_hbm, et_hbm, winv_hbm, zacc_hbm, out_hbm,
          winv_sh, acc_sh, rows_v, src_v, et_v, dst_v, gk_v, k_v, w_v,
          sem_i, sem_r0, sem_r1, sem_w0, sem_w1):
    cid = lax.axis_index("c")
    sid = lax.axis_index("s")
    wid = sid * NC + cid
    sem_r = (sem_r0, sem_r1)
    sem_w = (sem_w0, sem_w1)

    pltpu.sync_copy(zacc_hbm.at[pl.ds(sid * ASL, ASL)],
                    acc_sh.at[pl.ds(sid * ASL, ASL)])
    pltpu.sync_copy(winv_hbm.at[layer, 0, pl.ds(sid * DSL, DSL)],
                    winv_sh.at[pl.ds(sid * DSL, DSL)])
    plsc.subcore_barrier()

    mbase = wid * (ep // NW)

    def fetch_issue(g, s):
      eb = mbase + g * C
      c1 = pltpu.async_copy(src_hbm.at[pl.ds(eb, C)], src_v, sem_i)
      c2 = pltpu.async_copy(dst_hbm.at[pl.ds(eb, C)], dst_v.at[s], sem_i)
      c3 = pltpu.async_copy(et_hbm.at[pl.ds(eb, C)], et_v, sem_i)
      c1.wait()
      c2.wait()
      c3.wait()
      for j in range(C // 16):
        sl = pl.ds(j * 16, 16)
        rel = et_v[sl] * N
        gk_v[s, sl] = rel + src_v[sl]
        k_v[s, sl] = rel + dst_v[s, sl]
      pltpu.async_copy(y_hbm.at[gk_v.at[s]], rows_v.at[s], sem_r[s])
      pltpu.async_copy(winv_sh.at[k_v.at[s]], w_v.at[s], sem_w[s])

    def finish(s):
      pltpu.make_async_copy(y_hbm.at[gk_v.at[s]], rows_v.at[s],
                            sem_r[s]).wait()
      pltpu.make_async_copy(winv_sh.at[k_v.at[s]], w_v.at[s],
                            sem_w[s]).wait()
      def scale(gr, c2):
        w16 = w_v[s, pl.ds(gr * 16, 16)]
        for i in range(16):
          w = w16[i]
          for j in range(F // 16):
            sl = pl.ds(j * 16, 16)
            rows_v[s, gr * 16 + i, sl] = rows_v[s, gr * 16 + i, sl] * w
        return c2
      lax.fori_loop(0, C // 16, scale, 0)
      pltpu.sync_copy(rows_v.at[s], acc_sh.at[dst_v.at[s]], add=True)

    fetch_issue(0, 0)
    def pair(h, carry):
      g = 2 * h
      fetch_issue(g + 1, 1)
      finish(0)
      @pl.when(g + 2 < n_main)
      def _():
        fetch_issue(g + 2, 0)
      finish(1)
      return carry
    lax.fori_loop(0, n_main // 2, pair, 0)
    plsc.subcore_barrier()

    osl = pl.ds(sid * ASL, ASL)
    pltpu.sync_copy(acc_sh.at[osl], out_hbm.at[cid, osl])

  return agg


def _mm1(x, w):
  def body(x_ref, w_ref, o_ref):
    o_ref[0] = jnp.dot(x_ref[...], w_ref[0], preferred_element_type=jnp.float32)
  return pl.pallas_call(
      body,
      grid=(NREL, N // BN),
      in_specs=[pl.BlockSpec((BN, F), lambda r, n: (n, 0)),
                pl.BlockSpec((1, F, F), lambda r, n: (r, 0, 0))],
      out_specs=pl.BlockSpec((1, BN, F), lambda r, n: (r, n, 0)),
      out_shape=jax.ShapeDtypeStruct((NREL, N, F), jnp.float32),
  )(x, w)


def _mm2(hp, b1, w2):
  def body(hp_ref, b_ref, w_ref, o_ref):
    h = hp_ref[0] + hp_ref[1] + jnp.sum(b_ref[...], axis=0, keepdims=True)
    o_ref[0] = jnp.dot(h, w_ref[0], preferred_element_type=jnp.float32)
  return pl.pallas_call(
      body,
      grid=(NREL, N // BN),
      in_specs=[pl.BlockSpec((2, BN, F), lambda r, n: (0, n, 0)),
                pl.BlockSpec((NREL, F), lambda r, n: (0, 0)),
                pl.BlockSpec((1, F, F), lambda r, n: (r, 0, 0))],
      out_specs=pl.BlockSpec((1, BN, F), lambda r, n: (r, n, 0)),
      out_shape=jax.ShapeDtypeStruct((NREL, N, F), jnp.float32),
  )(hp, b1, w2)


def _final(op, b2):
  def body(op_ref, b_ref, o_ref):
    o_ref[...] = op_ref[0] + op_ref[1] + jnp.sum(b_ref[...], axis=0,
                                                 keepdims=True)
  return pl.pallas_call(
      body,
      grid=(N // BN,),
      in_specs=[pl.BlockSpec((2, BN, F), lambda n: (0, n, 0)),
                pl.BlockSpec((NREL, F), lambda n: (0, 0))],
      out_specs=pl.BlockSpec((BN, F), lambda n: (n, 0)),
      out_shape=jax.ShapeDtypeStruct((N, F), jnp.float32),
  )(op, b2)


def _pad_edges(src, dst, et, ep):
  p = ep - src.shape[0]
  i = jnp.arange(p, dtype=jnp.int32)
  src = jnp.concatenate([src, i % 4096])
  dst = jnp.concatenate([dst, N + (i % 112)])
  et = jnp.concatenate([et, jnp.full((p,), NREL - 1, jnp.int32)])
  return src, dst, et


def kernel(x, edge_index0, edge_type0, edge_index1, edge_type1, W1, b1,
           W2, b2):
  z_acc = jnp.zeros((NA, F), jnp.float32)
  z_deg = jnp.zeros((2 * RNP // NS,), jnp.float32)

  s0, d0, t0 = _pad_edges(edge_index0[0], edge_index0[1], edge_type0, EP0)
  s1, d1, t1 = _pad_edges(edge_index1[0], edge_index1[1], edge_type1, EP1)

  degs = _sc_degrees()(d0, t0, d1, t1, z_deg)
  winv = _winv(degs)

  y1 = _mm1(x, W1).reshape(RN, F)
  hp = _sc_edges(EP0, 0)(y1, s0, d0, t0, winv, z_acc)

  y2 = _mm2(hp, b1, W2).reshape(RN, F)
  op = _sc_edges(EP1, 1)(y2, s1, d1, t1, winv, z_acc)

  return _final(op, b2)


# speedup vs baseline: 17.0872x; 1.1032x over previous
import functools

import jax
import jax.numpy as jnp
from jax import lax
from jax.experimental import pallas as pl
from jax.experimental.pallas import tpu as pltpu
from jax.experimental.pallas import tpu_sc as plsc

N = 10000
F = 128
NREL = 3
RN = NREL * N
RNP = 30720
NA = 10112
NC = 2
NS = 16
NW = NC * NS
C = 112
CD = 336
BN = 1000
EP0 = 322560
EP1 = 161280
DSL = RNP // NS
ASL = NA // NS

_MESH = dict(core_axis_name="c", subcore_axis_name="s", num_cores=NC,
             num_subcores=NS)
_PARAMS = pltpu.CompilerParams(needs_layout_passes=False)


def _sc_degrees():
  zsl = 2 * RNP // NS

  @functools.partial(
      pl.kernel,
      out_type=jax.ShapeDtypeStruct((2, NC, 1, RNP), jnp.float32),
      mesh=plsc.VectorSubcoreMesh(**_MESH),
      compiler_params=_PARAMS,
      scratch_types=[
          pltpu.VMEM_SHARED((2 * RNP,), jnp.float32),
          pltpu.VMEM((CD,), jnp.int32),
          pltpu.VMEM((CD,), jnp.int32),
          pltpu.VMEM((CD,), jnp.int32),
          pltpu.VMEM((CD,), jnp.float32),
          pltpu.SemaphoreType.DMA,
      ],
  )
  def deg(d0_hbm, t0_hbm, d1_hbm, t1_hbm, zdeg_hbm, out_hbm,
          deg_sh, dd_v, de_v, dk_v, ones_v, sem):
    cid = lax.axis_index("c")
    sid = lax.axis_index("s")
    wid = sid * NC + cid

    for j in range(CD // 16):
      ones_v[pl.ds(j * 16, 16)] = jnp.ones((16,), jnp.float32)
    pltpu.sync_copy(zdeg_hbm, deg_sh.at[pl.ds(sid * zsl, zsl)])
    plsc.subcore_barrier()

    for l, (dh, th, epl) in enumerate(((d0_hbm, t0_hbm, EP0),
                                       (d1_hbm, t1_hbm, EP1))):
      base = wid * (epl // NW)
      off = l * RNP
      def chunk(g, carry):
        eb = base + g * CD
        c1 = pltpu.async_copy(dh.at[pl.ds(eb, CD)], dd_v, sem)
        c2 = pltpu.async_copy(th.at[pl.ds(eb, CD)], de_v, sem)
        c1.wait()
        c2.wait()
        for j in range(CD // 16):
          sl = pl.ds(j * 16, 16)
          dk_v[sl] = de_v[sl] * N + dd_v[sl] + off
        pltpu.sync_copy(ones_v, deg_sh.at[dk_v], add=True)
        return carry
      lax.fori_loop(0, epl // NW // CD, chunk, 0)
    plsc.subcore_barrier()

    for l in range(2):
      pltpu.sync_copy(deg_sh.at[pl.ds(l * RNP + sid * DSL, DSL)],
                      out_hbm.at[l, cid, 0, pl.ds(sid * DSL, DSL)])

  return deg


def _winv(degs):
  def body(d_ref, o_ref):
    o_ref[0, 0] = 1.0 / jnp.maximum(d_ref[0, 0, 0] + d_ref[0, 1, 0], 1.0)
  return pl.pallas_call(
      body,
      grid=(2,),
      in_specs=[pl.BlockSpec((1, NC, 1, RNP), lambda l: (l, 0, 0, 0))],
      out_specs=pl.BlockSpec((1, 1, RNP), lambda l: (l, 0, 0)),
      out_shape=jax.ShapeDtypeStruct((2, 1, RNP), jnp.float32),
  )(degs)


def _sc_edges(ep, layer):
  n_main = ep // NW // C
  ntrio = n_main // 3

  @functools.partial(
      pl.kernel,
      out_type=jax.ShapeDtypeStruct((NC, NA, F), jnp.float32),
      mesh=plsc.VectorSubcoreMesh(**_MESH),
      compiler_params=_PARAMS,
      scratch_types=[
          pltpu.VMEM_SHARED((RNP,), jnp.float32),
          pltpu.VMEM_SHARED((NA, F), jnp.float32),
          pltpu.VMEM((3, C, F), jnp.float32),
          pltpu.VMEM((C,), jnp.int32),
          pltpu.VMEM((C,), jnp.int32),
          pltpu.VMEM((3, C), jnp.int32),
          pltpu.VMEM((3, C), jnp.int32),
          pltpu.VMEM((3, C), jnp.int32),
          pltpu.VMEM((3, C), jnp.float32),
          pltpu.SemaphoreType.DMA,
          pltpu.SemaphoreType.DMA,
          pltpu.SemaphoreType.DMA,
          pltpu.SemaphoreType.DMA,
          pltpu.SemaphoreType.DMA,
          pltpu.SemaphoreType.DMA,
          pltpu.SemaphoreType.DMA,
          pltpu.SemaphoreType.DMA,
          pltpu.SemaphoreType.DMA,
          pltpu.SemaphoreType.DMA,
      ],
  )
  def agg(y_hbm, src_hbm, dst_hbm, et_hbm, winv_hbm, zacc_hbm, out_hbm,
          winv_sh, acc_sh, rows_v, src_v, et_v, dst_v, gk_v, k_v, w_v,
          sem_i, sem_r0, sem_r1, sem_r2, sem_w0, sem_w1, sem_w2,
          sem_s0, sem_s1, sem_s2):
    cid = lax.axis_index("c")
    sid = lax.axis_index("s")
    wid = sid * NC + cid
    sem_r = (sem_r0, sem_r1, sem_r2)
    sem_w = (sem_w0, sem_w1, sem_w2)
    sem_s = (sem_s0, sem_s1, sem_s2)

    pltpu.sync_copy(zacc_hbm.at[pl.ds(sid * ASL, ASL)],
                    acc_sh.at[pl.ds(sid * ASL, ASL)])
    pltpu.sync_copy(winv_hbm.at[layer, 0, pl.ds(sid * DSL, DSL)],
                    winv_sh.at[pl.ds(sid * DSL, DSL)])
    plsc.subcore_barrier()

    mbase = wid * (ep // NW)

    def drain_scatter(s):
      pltpu.make_async_copy(rows_v.at[s], acc_sh.at[dst_v.at[s]],
                            sem_s[s]).wait()

    def fetch_issue(g, s, drain):
      if drain:
        drain_scatter(s)
      eb = mbase + g * C
      c1 = pltpu.async_copy(src_hbm.at[pl.ds(eb, C)], src_v, sem_i)
      c2 = pltpu.async_copy(dst_hbm.at[pl.ds(eb, C)], dst_v.at[s], sem_i)
      c3 = pltpu.async_copy(et_hbm.at[pl.ds(eb, C)], et_v, sem_i)
      c1.wait()
      c2.wait()
      c3.wait()
      for j in range(C // 16):
        sl = pl.ds(j * 16, 16)
        rel = et_v[sl] * N
        gk_v[s, sl] = rel + src_v[sl]
        k_v[s, sl] = rel + dst_v[s, sl]
      pltpu.async_copy(y_hbm.at[gk_v.at[s]], rows_v.at[s], sem_r[s])
      pltpu.async_copy(winv_sh.at[k_v.at[s]], w_v.at[s], sem_w[s])

    def finish(s):
      pltpu.make_async_copy(y_hbm.at[gk_v.at[s]], rows_v.at[s],
                            sem_r[s]).wait()
      pltpu.make_async_copy(winv_sh.at[k_v.at[s]], w_v.at[s],
                            sem_w[s]).wait()
      def scale(gr, c2):
        w16 = w_v[s, pl.ds(gr * 16, 16)]
        for i in range(16):
          w = w16[i]
          for j in range(F // 16):
            sl = pl.ds(j * 16, 16)
            rows_v[s, gr * 16 + i, sl] = rows_v[s, gr * 16 + i, sl] * w
        return c2
      lax.fori_loop(0, C // 16, scale, 0)
      pltpu.async_copy(rows_v.at[s], acc_sh.at[dst_v.at[s]], sem_s[s],
                       add=True)

    fetch_issue(0, 0, False)
    fetch_issue(1, 1, False)
    finish(0)
    fetch_issue(2, 2, False)
    finish(1)
    fetch_issue(3, 0, True)
    finish(2)
    fetch_issue(4, 1, True)

    def trio(t, carry):
      g = 3 * t
      finish(0)
      fetch_issue(g + 2, 2, True)
      finish(1)
      @pl.when(g + 3 < n_main)
      def _():
        fetch_issue(g + 3, 0, True)
      finish(2)
      @pl.when(g + 4 < n_main)
      def _():
        fetch_issue(g + 4, 1, True)
      return carry
    lax.fori_loop(1, ntrio, trio, 0)

    for s in range(3):
      drain_scatter(s)
    plsc.subcore_barrier()

    osl = pl.ds(sid * ASL, ASL)
    pltpu.sync_copy(acc_sh.at[osl], out_hbm.at[cid, osl])

  return agg


def _mm1(x, w):
  def body(x_ref, w_ref, o_ref):
    o_ref[0] = jnp.dot(x_ref[...], w_ref[0], preferred_element_type=jnp.float32)
  return pl.pallas_call(
      body,
      grid=(NREL, N // BN),
      in_specs=[pl.BlockSpec((BN, F), lambda r, n: (n, 0)),
                pl.BlockSpec((1, F, F), lambda r, n: (r, 0, 0))],
      out_specs=pl.BlockSpec((1, BN, F), lambda r, n: (r, n, 0)),
      out_shape=jax.ShapeDtypeStruct((NREL, N, F), jnp.float32),
  )(x, w)


def _mm2(hp, b1, w2):
  def body(hp_ref, b_ref, w_ref, o_ref):
    h = hp_ref[0] + hp_ref[1] + jnp.sum(b_ref[...], axis=0, keepdims=True)
    o_ref[0] = jnp.dot(h, w_ref[0], preferred_element_type=jnp.float32)
  return pl.pallas_call(
      body,
      grid=(NREL, N // BN),
      in_specs=[pl.BlockSpec((2, BN, F), lambda r, n: (0, n, 0)),
                pl.BlockSpec((NREL, F), lambda r, n: (0, 0)),
                pl.BlockSpec((1, F, F), lambda r, n: (r, 0, 0))],
      out_specs=pl.BlockSpec((1, BN, F), lambda r, n: (r, n, 0)),
      out_shape=jax.ShapeDtypeStruct((NREL, N, F), jnp.float32),
  )(hp, b1, w2)


def _final(op, b2):
  def body(op_ref, b_ref, o_ref):
    o_ref[...] = op_ref[0] + op_ref[1] + jnp.sum(b_ref[...], axis=0,
                                                 keepdims=True)
  return pl.pallas_call(
      body,
      grid=(N // BN,),
      in_specs=[pl.BlockSpec((2, BN, F), lambda n: (0, n, 0)),
                pl.BlockSpec((NREL, F), lambda n: (0, 0))],
      out_specs=pl.BlockSpec((BN, F), lambda n: (n, 0)),
      out_shape=jax.ShapeDtypeStruct((N, F), jnp.float32),
  )(op, b2)


def _pad_edges(src, dst, et, ep):
  p = ep - src.shape[0]
  i = jnp.arange(p, dtype=jnp.int32)
  src = jnp.concatenate([src, i % 4096])
  dst = jnp.concatenate([dst, N + (i % 112)])
  et = jnp.concatenate([et, jnp.full((p,), NREL - 1, jnp.int32)])
  return src, dst, et


def kernel(x, edge_index0, edge_type0, edge_index1, edge_type1, W1, b1,
           W2, b2):
  z_acc = jnp.zeros((NA, F), jnp.float32)
  z_deg = jnp.zeros((2 * RNP // NS,), jnp.float32)

  s0, d0, t0 = _pad_edges(edge_index0[0], edge_index0[1], edge_type0, EP0)
  s1, d1, t1 = _pad_edges(edge_index1[0], edge_index1[1], edge_type1, EP1)

  degs = _sc_degrees()(d0, t0, d1, t1, z_deg)
  winv = _winv(degs)

  y1 = _mm1(x, W1).reshape(RN, F)
  hp = _sc_edges(EP0, 0)(y1, s0, d0, t0, winv, z_acc)

  y2 = _mm2(hp, b1, W2).reshape(RN, F)
  op = _sc_edges(EP1, 1)(y2, s1, d1, t1, winv, z_acc)

  return _final(op, b2)
